# baseline (speedup 1.0000x reference)
import jax
import jax.numpy as jnp
import numpy as np
from jax.experimental import pallas as pl

N_RAYS = 8192
N_PTS = 524288
GS = 160
K0_DIM = 12
VIEWPE = 4
WIDTH = 128
INTERVAL = 0.5
ALPHA_INIT = 1e-06
ACT_SHIFT = float(np.log(1.0 / (1.0 - ALPHA_INIT) - 1.0))
XYZ_MIN = -1.0
XYZ_MAX = 1.0


def _trilinear(grid, xyz):
    C, D, H, W = grid.shape
    u = (xyz - XYZ_MIN) / (XYZ_MAX - XYZ_MIN)
    px = u[:, 0] * (W - 1)
    py = u[:, 1] * (H - 1)
    pz = u[:, 2] * (D - 1)
    x0 = jnp.clip(jnp.floor(px), 0, W - 2).astype(jnp.int32)
    y0 = jnp.clip(jnp.floor(py), 0, H - 2).astype(jnp.int32)
    z0 = jnp.clip(jnp.floor(pz), 0, D - 2).astype(jnp.int32)
    fx = px - x0.astype(px.dtype)
    fy = py - y0.astype(py.dtype)
    fz = pz - z0.astype(pz.dtype)
    flat = grid.reshape(C, -1)
    def g(z, y, x):
        return jnp.take(flat, (z * H + y) * W + x, axis=1)
    out = (g(z0, y0, x0) * (1 - fz) * (1 - fy) * (1 - fx)
         + g(z0, y0, x0 + 1) * (1 - fz) * (1 - fy) * fx
         + g(z0, y0 + 1, x0) * (1 - fz) * fy * (1 - fx)
         + g(z0, y0 + 1, x0 + 1) * (1 - fz) * fy * fx
         + g(z0 + 1, y0, x0) * fz * (1 - fy) * (1 - fx)
         + g(z0 + 1, y0, x0 + 1) * fz * (1 - fy) * fx
         + g(z0 + 1, y0 + 1, x0) * fz * fy * (1 - fx)
         + g(z0 + 1, y0 + 1, x0 + 1) * fz * fy * fx)
    return out.T


def _alpha_body(d_ref, alpha_ref, log1m_ref):
    d = d_ref[...]
    e = jnp.exp(d + ACT_SHIFT)
    inv = jax.lax.rsqrt(1.0 + e)
    alpha = 1.0 - inv
    alpha_ref[...] = alpha
    log1m_ref[...] = jnp.log(jnp.clip(1.0 - alpha, 1e-10, 1.0))


def kernel(xyz, viewdirs, ray_id, density_grid, k0_grid, w0, b0, w1, b1, w2, b2):
    density = _trilinear(density_grid[0], xyz)[:, 0]
    d2 = density.reshape(N_PTS // 128, 128)
    alpha2, log1m2 = pl.pallas_call(
        _alpha_body,
        out_shape=(jax.ShapeDtypeStruct(d2.shape, jnp.float32),
                   jax.ShapeDtypeStruct(d2.shape, jnp.float32)),
    )(d2)
    alpha = alpha2.reshape(-1)
    log1m = log1m2.reshape(-1)
    cum = jnp.cumsum(log1m)
    ecs = jnp.concatenate([jnp.zeros((1,), log1m.dtype), cum[:-1]])
    seg_start = jnp.searchsorted(ray_id, jnp.arange(N_RAYS))
    T = jnp.exp(ecs - ecs[seg_start][ray_id])
    weights = alpha * T
    alphainv_last = jnp.exp(jax.ops.segment_sum(log1m, ray_id, num_segments=N_RAYS))
    k0 = _trilinear(k0_grid[0], xyz)
    vd = viewdirs[ray_id]
    freqs = (2.0 ** jnp.arange(VIEWPE)).astype(jnp.float32)
    ang = vd[:, :, None] * freqs
    vd_emb = jnp.concatenate([vd, jnp.sin(ang).reshape(vd.shape[0], -1), jnp.cos(ang).reshape(vd.shape[0], -1)], axis=-1)
    feat = jnp.concatenate([k0, vd_emb], axis=-1)
    h = jax.nn.relu(feat @ w0 + b0)
    h = jax.nn.relu(h @ w1 + b1)
    rgb = jax.nn.sigmoid(h @ w2 + b2)
    rgb_marched = jax.ops.segment_sum(weights[:, None] * rgb, ray_id, num_segments=N_RAYS) + alphainv_last[:, None] * 1.0
    return (rgb_marched, alphainv_last)
